# baseline (speedup 1.0000x reference)
import functools

import jax
import jax.numpy as jnp
from jax.experimental import pallas as pl

N = 50000
E = 800000
F_IN = 4
HID = 64
H = 2
D = H * HID
NUM_GRAPHS = 64
NUM_CLASSES = 5


def _gat_layer(x, src, dst, W, a_src, a_dst, b):
    n = x.shape[0]
    h = (x @ W).reshape(n, H, HID)
    alpha_src = (h * a_src[None, :, :]).sum(-1)
    alpha_dst = (h * a_dst[None, :, :]).sum(-1)
    e = alpha_src[src] + alpha_dst[dst]
    e = jax.nn.leaky_relu(e, 0.2)
    e_max = jax.ops.segment_max(e, dst, num_segments=n,
                                indices_are_sorted=True)
    e_max = jnp.where(jnp.isfinite(e_max), e_max, 0.0)
    p = jnp.exp(e - e_max[dst])
    denom = jax.ops.segment_sum(p, dst, num_segments=n,
                                indices_are_sorted=True)
    attn = p / (denom[dst] + 1e-16)
    msg = h[src] * attn[:, :, None]
    out = jax.ops.segment_sum(msg, dst, num_segments=n,
                              indices_are_sorted=True)
    return out.reshape(n, D) + b[None, :]


def _final_kernel(pooled_ref, w_ref, b_ref, o_ref):
    o_ref[...] = jax.nn.sigmoid(
        jnp.dot(pooled_ref[...], w_ref[...], preferred_element_type=jnp.float32)
        + b_ref[...]
    )


def kernel(x, edge_index, batch, W1, a_src1, a_dst1, b1, W2, a_src2, a_dst2, b2,
           W3, a_src3, a_dst3, b3, W4, a_src4, a_dst4, b4, lin_W, lin_b):
    n = x.shape[0]
    ar = jnp.arange(n, dtype=edge_index.dtype)
    src = jnp.concatenate([edge_index[0], ar])
    dst = jnp.concatenate([edge_index[1], ar])
    dst, perm = jax.lax.sort_key_val(dst, jnp.arange(dst.shape[0], dtype=jnp.int32))
    src = src[perm]

    h = _gat_layer(x, src, dst, W1, a_src1, a_dst1, b1)
    h = jax.nn.relu(h)
    h = _gat_layer(h, src, dst, W2, a_src2, a_dst2, b2)
    h = jax.nn.relu(h)
    h = _gat_layer(h, src, dst, W3, a_src3, a_dst3, b3)
    h = jax.nn.relu(h)
    h = _gat_layer(h, src, dst, W4, a_src4, a_dst4, b4)
    h = jax.nn.relu(h)

    sums = jax.ops.segment_sum(h, batch, num_segments=NUM_GRAPHS)
    counts = jnp.bincount(batch, length=NUM_GRAPHS).astype(jnp.float32)
    pooled = sums / jnp.maximum(counts, 1.0)[:, None]

    pad_b = jnp.zeros((8, NUM_CLASSES), jnp.float32) + lin_b[None, :]
    logits = pl.pallas_call(
        _final_kernel,
        out_shape=jax.ShapeDtypeStruct((NUM_GRAPHS, NUM_CLASSES), jnp.float32),
    )(pooled, lin_W, pad_b[:1])
    return logits


# speedup vs baseline: 37.0713x; 37.0713x over previous
import functools

import jax
import jax.numpy as jnp
from jax import lax
from jax.experimental import pallas as pl
from jax.experimental.pallas import tpu as pltpu
from jax.experimental.pallas import tpu_sc as plsc

N = 50000
E = 800000
FIN = 4
HID = 64
H = 2
D = 128
NG = 64
NCLS = 5

NW = 32
NR = 1568
NQ = 392
NQP = 400
NP = NW * NR
ET = E + N
CHP = 2048
NCHP = -(-ET // CHP)
EPAD = NCHP * CHP
SENT = 1 << 29
CAPQ = 12288
CHA = 1024
CB = 256
NEG = -1e30

_MESH = plsc.VectorSubcoreMesh(core_axis_name="c", subcore_axis_name="s")
_CP = pltpu.CompilerParams(needs_layout_passes=False)


def _wid():
    return lax.axis_index("s") * 2 + lax.axis_index("c")


def _io16():
    return lax.iota(jnp.int32, 16)


def _take(x, idx):
    return x.at[idx].get(mode="promise_in_bounds")


def _lrelu(x):
    return jnp.where(x > 0, x, x * 0.2)


def _cumsum16(x):
    io = _io16()
    for d in (1, 2, 4, 8):
        sh = _take(x, jnp.maximum(io - d, 0))
        x = x + jnp.where(io >= d, sh, 0)
    return x


def _seg_combine(keys, vals, is_max):
    io = _io16()
    ident = NEG if is_max else 0.0
    for d in (1, 2, 4, 8):
        sh = jnp.maximum(io - d, 0)
        ksh = _take(keys, sh)
        cond = (io >= d) & (ksh == keys)
        new = []
        for v in vals:
            vsh = _take(v, sh)
            vsh = jnp.where(cond, vsh, jnp.float32(ident))
            new.append(jnp.maximum(v, vsh) if is_max else v + vsh)
        vals = new
    nxt = _take(keys, jnp.minimum(io + 1, 15))
    last = (io == 15) | (nxt != keys)
    return vals, last


def _prep_body(src_hbm, dst_hbm, sb_hbm, lb_hbm, ke_hbm,
               srcc, dstc, sq0, sq1, sq2, sq3, lq0, lq1, lq2, lq3,
               cntb, sem):
    w = _wid()
    base = w * NR
    io = _io16()
    sqs = (sq0, sq1, sq2, sq3)
    lqs = (lq0, lq1, lq2, lq3)

    def chunk_body(ci, ptrs):
        pltpu.async_copy(src_hbm.at[pl.ds(ci * CHP, CHP)], srcc, sem).wait()
        pltpu.async_copy(dst_hbm.at[pl.ds(ci * CHP, CHP)], dstc, sem).wait()

        def vec_body(j, ptrs):
            dv = dstc[pl.ds(j * 16, 16)]
            sv = srcc[pl.ds(j * 16, 16)]
            lb = dv - base
            inr = (lb >= 0) & (lb < NR)
            cs_all = _cumsum16(jnp.where(inr, 1, 0))

            def do_append(p):
                out = []
                for q in range(4):
                    m = inr & (lb >= q * NQ) & (lb < (q + 1) * NQ)
                    cs = _cumsum16(jnp.where(m, 1, 0))
                    pos = jnp.minimum(p[q] + cs - 1, CAPQ - 1)
                    pos = jnp.where(m, pos, CAPQ + io)
                    plsc.store_scatter(sqs[q], [pos], sv)
                    plsc.store_scatter(lqs[q], [pos], lb - q * NQ)
                    out.append(jnp.minimum(p[q] + cs[15], CAPQ - 1))
                return tuple(out)

            return lax.cond(cs_all[15] > 0, do_append, lambda p: p, ptrs)

        return lax.fori_loop(0, CHP // 16, vec_body, ptrs)

    z = jnp.int32(0)
    ptrs = lax.fori_loop(0, NCHP, chunk_body, (z, z, z, z))

    cv = jnp.zeros((16,), jnp.int32)
    for q in range(4):
        cv = jnp.where(io == q, ptrs[q], cv)
    cntb[...] = cv
    pltpu.async_copy(cntb, ke_hbm.at[w], sem).wait()
    for q in range(4):
        pltpu.async_copy(sqs[q].at[pl.ds(0, CAPQ)], sb_hbm.at[w, q],
                         sem).wait()
        pltpu.async_copy(lqs[q].at[pl.ds(0, CAPQ)], lb_hbm.at[w, q],
                         sem).wait()


@functools.partial(
    pl.kernel,
    out_type=[
        jax.ShapeDtypeStruct((NW, 4, CAPQ), jnp.int32),
        jax.ShapeDtypeStruct((NW, 4, CAPQ), jnp.int32),
        jax.ShapeDtypeStruct((NW, 16), jnp.int32),
    ],
    mesh=_MESH,
    compiler_params=_CP,
    scratch_types=[
        pltpu.VMEM((CHP,), jnp.int32),
        pltpu.VMEM((CHP,), jnp.int32),
    ] + [pltpu.VMEM((CAPQ + 16,), jnp.int32)] * 8 + [
        pltpu.VMEM((16,), jnp.int32),
        pltpu.SemaphoreType.DMA,
    ],
)
def _prep(src_hbm, dst_hbm, sb_hbm, lb_hbm, ke_hbm, *scratch):
    _prep_body(src_hbm, dst_hbm, sb_hbm, lb_hbm, ke_hbm, *scratch)


def _attn_body(as2f_hbm, ad2f_hbm, sb_hbm, lb_hbm, ke_hbm,
               mif_hbm, ab0_hbm, ab1_hbm,
               ast, adl, m0t, m1t, s0t, s1t, srcc, lqc, pb0, pb1, kev, mi4,
               sem):
    w = _wid()
    base = w * NR
    io = _io16()

    pltpu.async_copy(as2f_hbm, ast, sem).wait()
    pltpu.async_copy(ad2f_hbm.at[pl.ds(base * 2, NR * 2)], adl, sem).wait()
    pltpu.async_copy(ke_hbm.at[w], kev, sem).wait()

    def init_body(i, _):
        sl = pl.ds(i * 16, 16)
        m0t[sl] = jnp.full((16,), NEG, jnp.float32)
        m1t[sl] = jnp.full((16,), NEG, jnp.float32)
        s0t[sl] = jnp.zeros((16,), jnp.float32)
        s1t[sl] = jnp.zeros((16,), jnp.float32)
        return 0

    lax.fori_loop(0, (NR + 16) // 16, init_body, 0)

    kv = kev[...]

    def edges(q, keq, per_vec, chunk_end):
        def chunk_body(ci, _):
            pltpu.async_copy(sb_hbm.at[w, q, pl.ds(ci * CHA, CHA)], srcc,
                             sem).wait()
            pltpu.async_copy(lb_hbm.at[w, q, pl.ds(ci * CHA, CHA)], lqc,
                             sem).wait()

            def vec_body(j, _):
                g = ci * CHA + j * 16
                valid = (io + g) < keq
                sl = pl.ds(j * 16, 16)
                sv = jnp.where(valid, srcc[sl], 0)
                lqv = jnp.where(valid, lqc[sl], 0)
                ldv = lqv + q * NQ
                as0 = plsc.load_gather(ast, [sv * 2])
                as1 = plsc.load_gather(ast, [sv * 2 + 1])
                ad0 = plsc.load_gather(adl, [ldv * 2])
                ad1 = plsc.load_gather(adl, [ldv * 2 + 1])
                e0 = _lrelu(as0 + ad0)
                e1 = _lrelu(as1 + ad1)
                per_vec(j, valid, ldv, e0, e1)
                return 0

            lax.fori_loop(0, CHA // 16, vec_body, 0)
            if chunk_end is not None:
                chunk_end(ci)
            return 0

        nch = (keq + CHA - 1) // CHA
        lax.fori_loop(0, nch, chunk_body, 0)

    for q in range(4):
        keq = kv[q]

        def max_vec(j, valid, ldv, e0, e1):
            e0 = jnp.where(valid, e0, jnp.float32(NEG))
            e1 = jnp.where(valid, e1, jnp.float32(NEG))
            ks, perm = plsc.sort_key_val(ldv, _io16())
            e0s = _take(e0, perm)
            e1s = _take(e1, perm)
            (g0, g1), last = _seg_combine(ks, [e0s, e1s], True)
            old0 = plsc.load_gather(m0t, [ks])
            old1 = plsc.load_gather(m1t, [ks])
            kw = jnp.where(last, ks, NR + _io16())
            plsc.store_scatter(m0t, [kw], jnp.maximum(old0, g0))
            plsc.store_scatter(m1t, [kw], jnp.maximum(old1, g1))

        edges(q, keq, max_vec, None)

        def sum_vec(j, valid, ldv, e0, e1):
            m0 = plsc.load_gather(m0t, [ldv])
            m1 = plsc.load_gather(m1t, [ldv])
            p0 = jnp.where(valid, jnp.exp(e0 - m0), 0.0)
            p1 = jnp.where(valid, jnp.exp(e1 - m1), 0.0)
            sl = pl.ds(j * 16, 16)
            pb0[sl] = p0
            pb1[sl] = p1
            ks, perm = plsc.sort_key_val(ldv, _io16())
            p0s = _take(p0, perm)
            p1s = _take(p1, perm)
            (g0, g1), last = _seg_combine(ks, [p0s, p1s], False)
            old0 = plsc.load_gather(s0t, [ks])
            old1 = plsc.load_gather(s1t, [ks])
            kw = jnp.where(last, ks, NR + _io16())
            plsc.store_scatter(s0t, [kw], old0 + g0)
            plsc.store_scatter(s1t, [kw], old1 + g1)

        def sum_chunk_end(ci, _q=q):
            pltpu.async_copy(pb0, ab0_hbm.at[w, _q, pl.ds(ci * CHA, CHA)],
                             sem).wait()
            pltpu.async_copy(pb1, ab1_hbm.at[w, _q, pl.ds(ci * CHA, CHA)],
                             sem).wait()

        edges(q, keq, sum_vec, sum_chunk_end)

    def fin_body(i, _):
        sl = pl.ds(i * 16, 16)
        rows = io + i * 16
        s0 = s0t[sl]
        s1 = s1t[sl]
        is0 = jnp.where(s0 > 0, 1.0 / s0, 0.0)
        is1 = jnp.where(s1 > 0, 1.0 / s1, 0.0)
        plsc.store_scatter(mi4, [rows * 4], m0t[sl])
        plsc.store_scatter(mi4, [rows * 4 + 1], m1t[sl])
        plsc.store_scatter(mi4, [rows * 4 + 2], is0)
        plsc.store_scatter(mi4, [rows * 4 + 3], is1)
        return 0

    lax.fori_loop(0, NR // 16, fin_body, 0)
    pltpu.async_copy(mi4, mif_hbm.at[pl.ds(base * 4, NR * 4)], sem).wait()


@functools.partial(
    pl.kernel,
    out_type=[
        jax.ShapeDtypeStruct((NP * 4,), jnp.float32),
        jax.ShapeDtypeStruct((NW, 4, CAPQ), jnp.float32),
        jax.ShapeDtypeStruct((NW, 4, CAPQ), jnp.float32),
    ],
    mesh=_MESH,
    compiler_params=_CP,
    scratch_types=[
        pltpu.VMEM((NP * 2,), jnp.float32),
        pltpu.VMEM((NR * 2,), jnp.float32),
        pltpu.VMEM((NR + 16,), jnp.float32),
        pltpu.VMEM((NR + 16,), jnp.float32),
        pltpu.VMEM((NR + 16,), jnp.float32),
        pltpu.VMEM((NR + 16,), jnp.float32),
        pltpu.VMEM((CHA,), jnp.int32),
        pltpu.VMEM((CHA,), jnp.int32),
        pltpu.VMEM((CHA,), jnp.float32),
        pltpu.VMEM((CHA,), jnp.float32),
        pltpu.VMEM((16,), jnp.int32),
        pltpu.VMEM((NR * 4,), jnp.float32),
        pltpu.SemaphoreType.DMA,
    ],
)
def _attn(as2f_hbm, ad2f_hbm, sb_hbm, lb_hbm, ke_hbm, mif_hbm, ab0_hbm,
          ab1_hbm, *scratch):
    _attn_body(as2f_hbm, ad2f_hbm, sb_hbm, lb_hbm, ke_hbm, mif_hbm, ab0_hbm,
               ab1_hbm, *scratch)


def _aggr_body(h_hbm, mif_hbm, sb_hbm, lb_hbm, ke_hbm, ab0_hbm, ab1_hbm,
               bias_hbm, hout_hbm,
               outl, rows, milf, srcc, lqc, ab0c, ab1c, bvecb, kev, sem,
               sem2):
    w = _wid()
    base = w * NR
    io = _io16()

    pltpu.async_copy(mif_hbm.at[pl.ds(base * 4, NR * 4)], milf, sem).wait()
    pltpu.async_copy(bias_hbm, bvecb, sem).wait()
    pltpu.async_copy(ke_hbm.at[w], kev, sem).wait()
    bv = [bvecb[pl.ds(k * 16, 16)] for k in range(8)]
    kv = kev[...]

    for q in range(4):
        keq = kv[q]

        def zero_body(n, _):
            for k in range(8):
                outl[n, pl.ds(k * 16, 16)] = jnp.zeros((16,), jnp.float32)
            return 0

        lax.fori_loop(0, NQP, zero_body, 0)

        def chunk_body(ci, _):
            pltpu.async_copy(sb_hbm.at[w, q, pl.ds(ci * CB, CB)], srcc,
                             sem).wait()
            pltpu.async_copy(lb_hbm.at[w, q, pl.ds(ci * CB, CB)], lqc,
                             sem).wait()
            cpa = pltpu.async_copy(ab0_hbm.at[w, q, pl.ds(ci * CB, CB)],
                                   ab0c, sem2)
            cpb = pltpu.async_copy(ab1_hbm.at[w, q, pl.ds(ci * CB, CB)],
                                   ab1c, sem2)

            def san_body(j, _):
                sl = pl.ds(j * 16, 16)
                valid = (io + (ci * CB + j * 16)) < keq
                srcc[sl] = jnp.where(valid, srcc[sl], 0)
                lqc[sl] = jnp.where(valid, lqc[sl], 0)
                return 0

            lax.fori_loop(0, CB // 16, san_body, 0)
            cpa.wait()
            cpb.wait()

            def san2_body(j, _):
                sl = pl.ds(j * 16, 16)
                valid = (io + (ci * CB + j * 16)) < keq
                ab0c[sl] = jnp.where(valid, ab0c[sl], 0.0)
                ab1c[sl] = jnp.where(valid, ab1c[sl], 0.0)
                return 0

            lax.fori_loop(0, CB // 16, san2_body, 0)

            pltpu.async_copy(h_hbm.at[srcc], rows, sem).wait()

            def grp_body(g, _):
                sl16 = pl.ds(g * 16, 16)
                lqv = lqc[sl16]
                a0v = ab0c[sl16]
                a1v = ab1c[sl16]
                for j in range(16):
                    lq = lqv[j]
                    va0 = jnp.full((16,), a0v[j], jnp.float32)
                    va1 = jnp.full((16,), a1v[j], jnp.float32)
                    for k in range(8):
                        va = va0 if k < 4 else va1
                        sl = pl.ds(k * 16, 16)
                        r = rows[g * 16 + j, sl]
                        outl[lq, sl] = outl[lq, sl] + va * r
                return 0

            lax.fori_loop(0, CB // 16, grp_body, 0)
            return 0

        nch = (keq + CB - 1) // CB
        lax.fori_loop(0, nch, chunk_body, 0)

        def fin_body(g, _):
            ldv = jnp.minimum(io + (g * 16 + q * NQ), NR - 1)
            i0v = plsc.load_gather(milf, [ldv * 4 + 2])
            i1v = plsc.load_gather(milf, [ldv * 4 + 3])
            for j in range(16):
                n = g * 16 + j
                vi0 = jnp.full((16,), i0v[j], jnp.float32)
                vi1 = jnp.full((16,), i1v[j], jnp.float32)
                for k in range(8):
                    vi = vi0 if k < 4 else vi1
                    sl = pl.ds(k * 16, 16)
                    o = outl[n, sl] * vi + bv[k]
                    outl[n, sl] = jnp.maximum(o, 0.0)
            return 0

        lax.fori_loop(0, NQP // 16, fin_body, 0)
        pltpu.async_copy(outl.at[pl.ds(0, NQ), :],
                         hout_hbm.at[pl.ds(base + q * NQ, NQ), :],
                         sem).wait()


@functools.partial(
    pl.kernel,
    out_type=[jax.ShapeDtypeStruct((NP, D), jnp.float32)],
    mesh=_MESH,
    compiler_params=_CP,
    scratch_types=[
        pltpu.VMEM((NQP, D), jnp.float32),
        pltpu.VMEM((CB, D), jnp.float32),
        pltpu.VMEM((NR * 4,), jnp.float32),
        pltpu.VMEM((CB,), jnp.int32),
        pltpu.VMEM((CB,), jnp.int32),
        pltpu.VMEM((CB,), jnp.float32),
        pltpu.VMEM((CB,), jnp.float32),
        pltpu.VMEM((D,), jnp.float32),
        pltpu.VMEM((16,), jnp.int32),
        pltpu.SemaphoreType.DMA,
        pltpu.SemaphoreType.DMA,
    ],
)
def _aggr(h_hbm, mif_hbm, sb_hbm, lb_hbm, ke_hbm, ab0_hbm, ab1_hbm, bias_hbm,
          hout_hbm, *scratch):
    _aggr_body(h_hbm, mif_hbm, sb_hbm, lb_hbm, ke_hbm, ab0_hbm, ab1_hbm,
               bias_hbm, hout_hbm, *scratch)


def _tc_layer_kernel(x_ref, w_ref, am_ref, h_ref, as_ref, ad_ref):
    h = jnp.dot(x_ref[...], w_ref[...], preferred_element_type=jnp.float32)
    h_ref[...] = h
    al = jnp.dot(h, am_ref[...], preferred_element_type=jnp.float32)
    as_ref[...] = al[:, 0:2]
    ad_ref[...] = al[:, 2:4]


def _tc_layer(x, W, amat):
    fin = x.shape[1]
    return pl.pallas_call(
        _tc_layer_kernel,
        grid=(NP // 128,),
        in_specs=[
            pl.BlockSpec((128, fin), lambda i: (i, 0)),
            pl.BlockSpec((fin, 128), lambda i: (0, 0)),
            pl.BlockSpec((128, 4), lambda i: (0, 0)),
        ],
        out_specs=[
            pl.BlockSpec((128, 128), lambda i: (i, 0)),
            pl.BlockSpec((128, 2), lambda i: (i, 0)),
            pl.BlockSpec((128, 2), lambda i: (i, 0)),
        ],
        out_shape=[
            jax.ShapeDtypeStruct((NP, 128), jnp.float32),
            jax.ShapeDtypeStruct((NP, 2), jnp.float32),
            jax.ShapeDtypeStruct((NP, 2), jnp.float32),
        ],
    )(x, W, amat)


def _tc_pool_kernel(h_ref, b_ref, lw_ref, lb_ref, praw_ref, cnt_ref, out_ref):
    i = pl.program_id(0)

    @pl.when(i == 0)
    def _():
        praw_ref[...] = jnp.zeros_like(praw_ref)
        cnt_ref[...] = jnp.zeros_like(cnt_ref)

    b = b_ref[0]
    gid = lax.broadcasted_iota(jnp.int32, (NG, 128), 0)
    oh = (gid == b).astype(jnp.float32)
    praw_ref[...] += jnp.dot(oh, h_ref[...],
                             preferred_element_type=jnp.float32)
    cnt_ref[...] += jnp.dot(oh, jnp.ones((128, 128), jnp.float32),
                            preferred_element_type=jnp.float32)

    @pl.when(i == NP // 128 - 1)
    def _():
        pooled = praw_ref[...] / jnp.maximum(cnt_ref[...], 1.0)
        lg = jnp.dot(pooled, lw_ref[...],
                     preferred_element_type=jnp.float32) + lb_ref[0:1, :]
        out_ref[...] = 1.0 / (1.0 + jnp.exp(-lg))


def _tc_pool(h, batch_rows, lw_pad, lb_pad):
    return pl.pallas_call(
        _tc_pool_kernel,
        grid=(NP // 128,),
        in_specs=[
            pl.BlockSpec((128, 128), lambda i: (i, 0)),
            pl.BlockSpec((1, 1, 128), lambda i: (i, 0, 0)),
            pl.BlockSpec((128, 128), lambda i: (0, 0)),
            pl.BlockSpec((8, 128), lambda i: (0, 0)),
        ],
        out_specs=[
            pl.BlockSpec((NG, 128), lambda i: (0, 0)),
            pl.BlockSpec((NG, 128), lambda i: (0, 0)),
            pl.BlockSpec((NG, 128), lambda i: (0, 0)),
        ],
        out_shape=[
            jax.ShapeDtypeStruct((NG, 128), jnp.float32),
            jax.ShapeDtypeStruct((NG, 128), jnp.float32),
            jax.ShapeDtypeStruct((NG, 128), jnp.float32),
        ],
    )(h, batch_rows, lw_pad, lb_pad)


def _amat(a_src, a_dst):
    z = jnp.zeros((HID,), jnp.float32)
    cols = [
        jnp.concatenate([a_src[0], z]),
        jnp.concatenate([z, a_src[1]]),
        jnp.concatenate([a_dst[0], z]),
        jnp.concatenate([z, a_dst[1]]),
    ]
    return jnp.stack(cols, axis=1)


def kernel(x, edge_index, batch, W1, a_src1, a_dst1, b1, W2, a_src2, a_dst2,
           b2, W3, a_src3, a_dst3, b3, W4, a_src4, a_dst4, b4, lin_W, lin_b):
    ar = jnp.arange(N, dtype=jnp.int32)
    src = jnp.concatenate([edge_index[0], ar])
    dst = jnp.concatenate([edge_index[1], ar])
    src = jnp.pad(src, (0, EPAD - ET))
    dst = jnp.pad(dst, (0, EPAD - ET), constant_values=SENT)

    sb, lb, ke = _prep(src, dst)

    xp = jnp.pad(x, ((0, NP - N), (0, 0)))
    h = xp
    for W, a_s, a_d, b in ((W1, a_src1, a_dst1, b1), (W2, a_src2, a_dst2, b2),
                           (W3, a_src3, a_dst3, b3), (W4, a_src4, a_dst4, b4)):
        h_mm, as2, ad2 = _tc_layer(h, W, _amat(a_s, a_d))
        mif, ab0, ab1 = _attn(as2.reshape(NP * 2), ad2.reshape(NP * 2),
                              sb, lb, ke)
        (h,) = _aggr(h_mm, mif, sb, lb, ke, ab0, ab1, b)

    batch_rows = jnp.pad(batch.astype(jnp.int32), (0, NP - N),
                         constant_values=NG).reshape(NP // 128, 1, 128)
    lw_pad = jnp.pad(lin_W, ((0, 0), (0, 128 - NCLS)))
    lb_pad = jnp.zeros((8, 128), jnp.float32).at[:, :NCLS].set(lin_b[None, :])
    _, _, out_full = _tc_pool(h, batch_rows, lw_pad, lb_pad)
    return out_full[:NG, :NCLS]
